# baseline (speedup 1.0000x reference)
import jax
import jax.numpy as jnp
from jax import lax
from jax.experimental import pallas as pl
from jax.experimental.pallas import tpu as pltpu
from jax.experimental.pallas import tpu_sc as plsc

NC = 2
NS = 16
L = 16
NW = NC * NS

N = 32768
K = N // 4
ROWS = 2048
ROWS_PER_W = ROWS // NW

MININT = -(2**31)


def _desc_key(v):
    u = lax.bitcast_convert_type(v, jnp.int32)
    m = lax.shift_right_arithmetic(u, 31)
    a = u ^ (m | MININT)
    return ~a


def _key_to_f32(k):
    a = ~k
    u = jnp.where(a < 0, a ^ MININT, ~a)
    return lax.bitcast_convert_type(u, jnp.float32)


def _sc_body(x_hbm, o_hbm, row_v, a_v, b_v, hist_v, off_v, out_v, sem):
    wid = lax.axis_index("s") * NC + lax.axis_index("c")
    lane = lax.iota(jnp.int32, L)
    ones = jnp.ones((L,), jnp.int32)
    zeros = jnp.zeros((L,), jnp.int32)
    fill = jnp.full((L,), -1, jnp.int32)

    def do_row(r, _):
        row = wid * ROWS_PER_W + r
        pltpu.sync_copy(x_hbm.at[row], row_v)

        def zero_hist(i, _):
            hist_v[pl.ds(i * L, L)] = zeros
            return 0
        lax.fori_loop(0, 256, zero_hist, 0, unroll=8)

        def h1(i, _):
            dk = _desc_key(row_v[pl.ds(i * L, L)])
            d = lax.shift_right_logical(dk, 24)
            plsc.addupdate_scatter(hist_v, [d * L + lane], ones)
            return 0
        lax.fori_loop(0, N // L, h1, 0, unroll=8)

        def coarse_sum(j, _):
            acc = zeros
            for jj in range(16):
                acc = acc + hist_v[pl.ds((j * 16 + jj) * L, L)]
            off_v[pl.ds(j * L, L)] = acc
            return 0
        lax.fori_loop(0, 16, coarse_sum, 0, unroll=2)

        def scan_c(j, carry):
            cum, jstar, cbase = carry
            t = jnp.sum(off_v[pl.ds(j * L, L)])
            ncum = cum + t
            crossed = jnp.logical_and(cum < K, ncum >= K)
            jstar = jnp.where(crossed, j, jstar)
            cbase = jnp.where(crossed, cum, cbase)
            return ncum, jstar, cbase
        _, jstar, cbase = lax.fori_loop(
            0, 16, scan_c, (jnp.int32(0), jnp.int32(0), jnp.int32(0)))

        def scan_f(i, carry):
            cum, dstar = carry
            d = jstar * 16 + i
            t = jnp.sum(hist_v[pl.ds(d * L, L)])
            ncum = cum + t
            crossed = jnp.logical_and(cum < K, ncum >= K)
            dstar = jnp.where(crossed, d, dstar)
            return ncum, dstar
        _, dstar = lax.fori_loop(0, 16, scan_f, (cbase, jnp.int32(0)))

        def keep_scan(d, acc):
            h = hist_v[pl.ds(d * L, L)]
            hist_v[pl.ds(d * L, L)] = zeros
            return acc + h * (d <= dstar).astype(jnp.int32)
        hkeep = lax.fori_loop(0, 256, keep_scan, zeros, unroll=4)
        base = plsc.cumsum(hkeep) - hkeep
        m_cnt = jnp.sum(hkeep)

        def compact(i, offv):
            dk = _desc_key(row_v[pl.ds(i * L, L)])
            d = lax.shift_right_logical(dk, 24)
            msk = d <= dstar
            plsc.store_scatter(a_v, [offv], dk, mask=msk)
            return offv + msk.astype(jnp.int32)
        lax.fori_loop(0, N // L, compact, base, unroll=8)
        for jj in range(4):
            a_v[pl.ds(m_cnt + jj * L, L)] = fill
        t_blk = (lax.shift_right_logical(m_cnt + (L - 1), 4) + 3) & ~3
        t_4 = lax.shift_right_logical(t_blk, 2)
        lane_t = lane * t_blk

        def hist_pass(src, shift):
            def hp(ii, _):
                i0 = ii * 4
                for jj in range(4):
                    k = plsc.load_gather(src, [lane_t + (i0 + jj)])
                    d = lax.shift_right_logical(k, shift) & 255
                    plsc.addupdate_scatter(hist_v, [d * L + lane], ones)
                return 0
            lax.fori_loop(0, t_4, hp, 0)

        def offs_pass():
            def offs(d, carry):
                h = hist_v[pl.ds(d * L, L)]
                hist_v[pl.ds(d * L, L)] = zeros
                incl = plsc.cumsum(h)
                off_v[pl.ds(d * L, L)] = incl - h + carry
                return carry + jnp.sum(h)
            lax.fori_loop(0, 256, offs, jnp.int32(0), unroll=4)

        def perm_pass(src, dst, shift):
            def perm(ii, _):
                i0 = ii * 4
                for jj in range(4):
                    k = plsc.load_gather(src, [lane_t + (i0 + jj)])
                    d = lax.shift_right_logical(k, shift) & 255
                    oidx = d * L + lane
                    o = plsc.load_gather(off_v, [oidx])
                    plsc.store_scatter(dst, [o], k)
                    plsc.store_scatter(off_v, [oidx], o + 1)
                return 0
            lax.fori_loop(0, t_4, perm, 0)

        hist_pass(a_v, 8)
        offs_pass()
        perm_pass(a_v, b_v, 8)
        hist_pass(b_v, 16)
        offs_pass()
        perm_pass(b_v, a_v, 16)
        hist_pass(a_v, 24)
        offs_pass()

        def permf(ii, _):
            i0 = ii * 4
            for jj in range(4):
                k = plsc.load_gather(a_v, [lane_t + (i0 + jj)])
                d = lax.shift_right_logical(k, 24)
                oidx = d * L + lane
                o = plsc.load_gather(off_v, [oidx])
                plsc.store_scatter(out_v, [o], _key_to_f32(k), mask=o < K)
                plsc.store_scatter(off_v, [oidx], o + 1)
            return 0
        lax.fori_loop(0, t_4, permf, 0)

        pltpu.sync_copy(out_v, o_hbm.at[row])
        return 0

    lax.fori_loop(0, ROWS_PER_W, do_row, 0)


@jax.jit
def kernel(input):
    b, h, n = input.shape
    x = input.reshape(ROWS, N)
    out = pl.kernel(
        _sc_body,
        out_type=jax.ShapeDtypeStruct((ROWS, K), jnp.float32),
        mesh=plsc.VectorSubcoreMesh(core_axis_name="c", subcore_axis_name="s"),
        compiler_params=pltpu.CompilerParams(needs_layout_passes=False),
        scratch_types=[
            pltpu.VMEM((N,), jnp.float32),
            pltpu.VMEM((N + 4 * L,), jnp.int32),
            pltpu.VMEM((N + 4 * L,), jnp.int32),
            pltpu.VMEM((4096,), jnp.int32),
            pltpu.VMEM((4096,), jnp.int32),
            pltpu.VMEM((K,), jnp.float32),
            pltpu.SemaphoreType.DMA,
        ],
    )(x)
    return out.reshape(b, h, K)


# speedup vs baseline: 1.8207x; 1.8207x over previous
import jax
import jax.numpy as jnp
from jax import lax
from jax.experimental import pallas as pl
from jax.experimental.pallas import tpu as pltpu
from jax.experimental.pallas import tpu_sc as plsc

NC = 2
NS = 16
L = 16
NW = NC * NS

N = 32768
K = N // 4
ROWS = 2048
ROWS_PER_W = ROWS // NW

MININT = -(2**31)


def _desc_key(v):
    u = lax.bitcast_convert_type(v, jnp.int32)
    m = lax.shift_right_arithmetic(u, 31)
    a = u ^ (m | MININT)
    return ~a


def _key_to_f32(k):
    a = ~k
    u = jnp.where(a < 0, a ^ MININT, ~a)
    return lax.bitcast_convert_type(u, jnp.float32)


def _sc_body(x_hbm, o_hbm, row_v, a_v, b_v, hist_v, off_v, out_v, sem):
    wid = lax.axis_index("s") * NC + lax.axis_index("c")
    lane = lax.iota(jnp.int32, L)
    ones = jnp.ones((L,), jnp.int32)
    zeros = jnp.zeros((L,), jnp.int32)
    fill = jnp.full((L,), -1, jnp.int32)

    def do_row(r, _):
        row = wid * ROWS_PER_W + r
        pltpu.sync_copy(x_hbm.at[row], row_v)

        def zero_hist(i, _):
            hist_v[pl.ds(i * L, L)] = zeros
            return 0
        lax.fori_loop(0, 256, zero_hist, 0, unroll=8)

        def h1_idx(i):
            dk = _desc_key(row_v[pl.ds(i * L, L)])
            d = lax.shift_right_logical(dk, 24)
            return d * L + lane

        def h1(i, oidx):
            nxt = h1_idx(i + 1)
            plsc.addupdate_scatter(hist_v, [oidx], ones)
            return nxt
        oidx_l = lax.fori_loop(0, N // L - 1, h1, h1_idx(0))
        plsc.addupdate_scatter(hist_v, [oidx_l], ones)

        def coarse_sum(j, _):
            acc = zeros
            for jj in range(16):
                acc = acc + hist_v[pl.ds((j * 16 + jj) * L, L)]
            off_v[pl.ds(j * L, L)] = acc
            return 0
        lax.fori_loop(0, 16, coarse_sum, 0, unroll=2)

        def scan_c(j, carry):
            cum, jstar, cbase = carry
            t = jnp.sum(off_v[pl.ds(j * L, L)])
            ncum = cum + t
            crossed = jnp.logical_and(cum < K, ncum >= K)
            jstar = jnp.where(crossed, j, jstar)
            cbase = jnp.where(crossed, cum, cbase)
            return ncum, jstar, cbase
        _, jstar, cbase = lax.fori_loop(
            0, 16, scan_c, (jnp.int32(0), jnp.int32(0), jnp.int32(0)))

        def scan_f(i, carry):
            cum, dstar = carry
            d = jstar * 16 + i
            t = jnp.sum(hist_v[pl.ds(d * L, L)])
            ncum = cum + t
            crossed = jnp.logical_and(cum < K, ncum >= K)
            dstar = jnp.where(crossed, d, dstar)
            return ncum, dstar
        _, dstar = lax.fori_loop(0, 16, scan_f, (cbase, jnp.int32(0)))

        def keep_scan(d, acc):
            h = hist_v[pl.ds(d * L, L)]
            hist_v[pl.ds(d * L, L)] = zeros
            return acc + h * (d <= dstar).astype(jnp.int32)
        hkeep = lax.fori_loop(0, 256, keep_scan, zeros, unroll=4)
        base = plsc.cumsum(hkeep) - hkeep
        m_cnt = jnp.sum(hkeep)

        def key_msk(i):
            dk = _desc_key(row_v[pl.ds(i * L, L)])
            d = lax.shift_right_logical(dk, 24)
            return dk, d <= dstar

        def compact(i, carry):
            offv, dk, msk = carry
            dk_n, msk_n = key_msk(i + 1)
            plsc.store_scatter(a_v, [offv], dk, mask=msk)
            return offv + msk.astype(jnp.int32), dk_n, msk_n
        dk0, msk0 = key_msk(0)
        offv, dk_l, msk_l = lax.fori_loop(
            0, N // L - 1, compact, (base, dk0, msk0))
        plsc.store_scatter(a_v, [offv], dk_l, mask=msk_l)
        a_v[pl.ds(m_cnt, L)] = fill
        t_blk = lax.shift_right_logical(m_cnt + (L - 1), 4)
        lane_t = lane * t_blk

        def oidx_of(src, i, shift):
            k = plsc.load_gather(src, [lane_t + i])
            d = lax.shift_right_logical(k, shift) & 255
            return k, d * L + lane

        def hist_pass(src, shift):
            def hp(i, carry):
                _, oidx = carry
                nxt = oidx_of(src, i + 1, shift)
                plsc.addupdate_scatter(hist_v, [oidx], ones)
                return nxt
            _, oidx_l = lax.fori_loop(0, t_blk - 1, hp, oidx_of(src, 0, shift))
            plsc.addupdate_scatter(hist_v, [oidx_l], ones)

        def offs_pass():
            def offs(d, carry):
                h = hist_v[pl.ds(d * L, L)]
                hist_v[pl.ds(d * L, L)] = zeros
                incl = plsc.cumsum(h)
                off_v[pl.ds(d * L, L)] = incl - h + carry
                return carry + jnp.sum(h)
            lax.fori_loop(0, 256, offs, jnp.int32(0), unroll=4)

        def perm_pass(src, dst, shift):
            def emit(k, oidx):
                o = plsc.load_gather(off_v, [oidx])
                plsc.store_scatter(off_v, [oidx], o + 1)
                plsc.store_scatter(dst, [o], k)

            def perm(i, carry):
                k, oidx = carry
                nxt = oidx_of(src, i + 1, shift)
                emit(k, oidx)
                return nxt
            k_l, oidx_l = lax.fori_loop(0, t_blk - 1, perm,
                                        oidx_of(src, 0, shift))
            emit(k_l, oidx_l)

        hist_pass(a_v, 8)
        offs_pass()
        perm_pass(a_v, b_v, 8)
        hist_pass(b_v, 16)
        offs_pass()
        perm_pass(b_v, a_v, 16)
        hist_pass(a_v, 24)
        offs_pass()

        def emitf(k, oidx):
            o = plsc.load_gather(off_v, [oidx])
            plsc.store_scatter(off_v, [oidx], o + 1)
            plsc.store_scatter(out_v, [o], _key_to_f32(k), mask=o < K)

        def permf(i, carry):
            k, oidx = carry
            nxt = oidx_of(a_v, i + 1, 24)
            emitf(k, oidx)
            return nxt
        k_l, oidx_l = lax.fori_loop(0, t_blk - 1, permf, oidx_of(a_v, 0, 24))
        emitf(k_l, oidx_l)

        pltpu.sync_copy(out_v, o_hbm.at[row])
        return 0

    lax.fori_loop(0, ROWS_PER_W, do_row, 0)


@jax.jit
def kernel(input):
    b, h, n = input.shape
    x = input.reshape(ROWS, N)
    out = pl.kernel(
        _sc_body,
        out_type=jax.ShapeDtypeStruct((ROWS, K), jnp.float32),
        mesh=plsc.VectorSubcoreMesh(core_axis_name="c", subcore_axis_name="s"),
        compiler_params=pltpu.CompilerParams(needs_layout_passes=False),
        scratch_types=[
            pltpu.VMEM((N,), jnp.float32),
            pltpu.VMEM((N + 4 * L,), jnp.int32),
            pltpu.VMEM((N + 4 * L,), jnp.int32),
            pltpu.VMEM((4096,), jnp.int32),
            pltpu.VMEM((4096,), jnp.int32),
            pltpu.VMEM((K,), jnp.float32),
            pltpu.SemaphoreType.DMA,
        ],
    )(x)
    return out.reshape(b, h, K)


# speedup vs baseline: 1.8831x; 1.0342x over previous
import jax
import jax.numpy as jnp
from jax import lax
from jax.experimental import pallas as pl
from jax.experimental.pallas import tpu as pltpu
from jax.experimental.pallas import tpu_sc as plsc

NC = 2
NS = 16
L = 16
NW = NC * NS

N = 32768
K = N // 4
ROWS = 2048
ROWS_PER_W = ROWS // NW

MININT = -(2**31)


def _desc_key(v):
    u = lax.bitcast_convert_type(v, jnp.int32)
    m = lax.shift_right_arithmetic(u, 31)
    a = u ^ (m | MININT)
    return ~a


def _key_to_f32(k):
    a = ~k
    u = jnp.where(a < 0, a ^ MININT, ~a)
    return lax.bitcast_convert_type(u, jnp.float32)


def _sc_body(x_hbm, o_hbm, row_v, a_v, b_v, hist_v, off_v, out_v, sem):
    wid = lax.axis_index("s") * NC + lax.axis_index("c")
    lane = lax.iota(jnp.int32, L)
    ones = jnp.ones((L,), jnp.int32)
    zeros = jnp.zeros((L,), jnp.int32)
    fill = jnp.full((L,), -1, jnp.int32)

    def do_row(r, _):
        row = wid * ROWS_PER_W + r
        pltpu.sync_copy(x_hbm.at[row], row_v)

        def zero_hist(i, _):
            hist_v[pl.ds(i * L, L)] = zeros
            return 0
        lax.fori_loop(0, 256, zero_hist, 0, unroll=8)

        def h1_idx(i):
            dk = _desc_key(row_v[pl.ds(i * L, L)])
            d = lax.shift_right_logical(dk, 24)
            return d * L + lane

        def h1(i, oidx):
            nxt = h1_idx(i + 1)
            plsc.addupdate_scatter(hist_v, [oidx], ones)
            return nxt
        oidx_l = lax.fori_loop(0, N // L - 1, h1, h1_idx(0), unroll=4)
        plsc.addupdate_scatter(hist_v, [oidx_l], ones)

        def coarse_sum(j, _):
            acc = zeros
            for jj in range(16):
                acc = acc + hist_v[pl.ds((j * 16 + jj) * L, L)]
            off_v[pl.ds(j * L, L)] = acc
            return 0
        lax.fori_loop(0, 16, coarse_sum, 0, unroll=2)

        def scan_c(j, carry):
            cum, jstar, cbase = carry
            t = jnp.sum(off_v[pl.ds(j * L, L)])
            ncum = cum + t
            crossed = jnp.logical_and(cum < K, ncum >= K)
            jstar = jnp.where(crossed, j, jstar)
            cbase = jnp.where(crossed, cum, cbase)
            return ncum, jstar, cbase
        _, jstar, cbase = lax.fori_loop(
            0, 16, scan_c, (jnp.int32(0), jnp.int32(0), jnp.int32(0)))

        def scan_f(i, carry):
            cum, dstar = carry
            d = jstar * 16 + i
            t = jnp.sum(hist_v[pl.ds(d * L, L)])
            ncum = cum + t
            crossed = jnp.logical_and(cum < K, ncum >= K)
            dstar = jnp.where(crossed, d, dstar)
            return ncum, dstar
        _, dstar = lax.fori_loop(0, 16, scan_f, (cbase, jnp.int32(0)))

        def keep_scan(d, acc):
            h = hist_v[pl.ds(d * L, L)]
            hist_v[pl.ds(d * L, L)] = zeros
            return acc + h * (d <= dstar).astype(jnp.int32)
        hkeep = lax.fori_loop(0, 256, keep_scan, zeros, unroll=4)
        base = plsc.cumsum(hkeep) - hkeep
        m_cnt = jnp.sum(hkeep)

        def key_msk(i):
            dk = _desc_key(row_v[pl.ds(i * L, L)])
            d = lax.shift_right_logical(dk, 24)
            return dk, d <= dstar

        def compact(i, carry):
            offv, dk, msk = carry
            dk_n, msk_n = key_msk(i + 1)
            plsc.store_scatter(a_v, [offv], dk, mask=msk)
            return offv + msk.astype(jnp.int32), dk_n, msk_n
        dk0, msk0 = key_msk(0)
        offv, dk_l, msk_l = lax.fori_loop(
            0, N // L - 1, compact, (base, dk0, msk0), unroll=2)
        plsc.store_scatter(a_v, [offv], dk_l, mask=msk_l)
        a_v[pl.ds(m_cnt, L)] = fill
        t_blk = lax.shift_right_logical(m_cnt + (L - 1), 4)
        lane_t = lane * t_blk

        def oidx_of(src, i, shift):
            k = plsc.load_gather(src, [lane_t + i])
            d = lax.shift_right_logical(k, shift) & 255
            return k, d * L + lane

        def hist_pass(src, shift):
            def hp(i, carry):
                _, oidx = carry
                nxt = oidx_of(src, i + 1, shift)
                plsc.addupdate_scatter(hist_v, [oidx], ones)
                return nxt
            _, oidx_l = lax.fori_loop(0, t_blk - 1, hp, oidx_of(src, 0, shift))
            plsc.addupdate_scatter(hist_v, [oidx_l], ones)

        def offs_pass():
            def offs(d, carry):
                h = hist_v[pl.ds(d * L, L)]
                hist_v[pl.ds(d * L, L)] = zeros
                incl = plsc.cumsum(h)
                off_v[pl.ds(d * L, L)] = incl - h + carry
                return carry + jnp.sum(h)
            lax.fori_loop(0, 256, offs, jnp.int32(0), unroll=8)

        def perm_pass(src, dst, shift):
            def emit(k, oidx):
                o = plsc.load_gather(off_v, [oidx])
                plsc.store_scatter(off_v, [oidx], o + 1)
                plsc.store_scatter(dst, [o], k)

            def perm(i, carry):
                k, oidx = carry
                nxt = oidx_of(src, i + 1, shift)
                emit(k, oidx)
                return nxt
            k_l, oidx_l = lax.fori_loop(0, t_blk - 1, perm,
                                        oidx_of(src, 0, shift))
            emit(k_l, oidx_l)

        hist_pass(a_v, 8)
        offs_pass()
        perm_pass(a_v, b_v, 8)
        hist_pass(b_v, 16)
        offs_pass()
        perm_pass(b_v, a_v, 16)
        hist_pass(a_v, 24)
        offs_pass()

        def emitf(k, oidx):
            o = plsc.load_gather(off_v, [oidx])
            plsc.store_scatter(off_v, [oidx], o + 1)
            plsc.store_scatter(out_v, [o], _key_to_f32(k), mask=o < K)

        def permf(i, carry):
            k, oidx = carry
            nxt = oidx_of(a_v, i + 1, 24)
            emitf(k, oidx)
            return nxt
        k_l, oidx_l = lax.fori_loop(0, t_blk - 1, permf, oidx_of(a_v, 0, 24))
        emitf(k_l, oidx_l)

        pltpu.sync_copy(out_v, o_hbm.at[row])
        return 0

    lax.fori_loop(0, ROWS_PER_W, do_row, 0)


@jax.jit
def kernel(input):
    b, h, n = input.shape
    x = input.reshape(ROWS, N)
    out = pl.kernel(
        _sc_body,
        out_type=jax.ShapeDtypeStruct((ROWS, K), jnp.float32),
        mesh=plsc.VectorSubcoreMesh(core_axis_name="c", subcore_axis_name="s"),
        compiler_params=pltpu.CompilerParams(needs_layout_passes=False),
        scratch_types=[
            pltpu.VMEM((N,), jnp.float32),
            pltpu.VMEM((N + 4 * L,), jnp.int32),
            pltpu.VMEM((N + 4 * L,), jnp.int32),
            pltpu.VMEM((4096,), jnp.int32),
            pltpu.VMEM((4096,), jnp.int32),
            pltpu.VMEM((K,), jnp.float32),
            pltpu.SemaphoreType.DMA,
        ],
    )(x)
    return out.reshape(b, h, K)


# speedup vs baseline: 1.9466x; 1.0337x over previous
import jax
import jax.numpy as jnp
from jax import lax
from jax.experimental import pallas as pl
from jax.experimental.pallas import tpu as pltpu
from jax.experimental.pallas import tpu_sc as plsc

NC = 2
NS = 16
L = 16
NW = NC * NS

N = 32768
K = N // 4
ROWS = 2048
ROWS_PER_W = ROWS // NW

MININT = -(2**31)


def _desc_key(v):
    u = lax.bitcast_convert_type(v, jnp.int32)
    m = lax.shift_right_arithmetic(u, 31)
    a = u ^ (m | MININT)
    return ~a


def _key_to_f32(k):
    a = ~k
    u = jnp.where(a < 0, a ^ MININT, ~a)
    return lax.bitcast_convert_type(u, jnp.float32)


def _sc_body(x_hbm, o_hbm, row0_v, row1_v, a_v, hist_v, off_v, out_v,
             sem0, sem1):
    wid = lax.axis_index("s") * NC + lax.axis_index("c")
    lane = lax.iota(jnp.int32, L)
    ones = jnp.ones((L,), jnp.int32)
    zeros = jnp.zeros((L,), jnp.int32)
    fill = jnp.full((L,), -1, jnp.int32)
    base_row = wid * ROWS_PER_W

    def copy_in(r, buf, sem):
        return pltpu.make_async_copy(
            x_hbm.at[base_row + r], buf.at[pl.ds(0, N)], sem)

    def compute_row(row, row_v):

        def zero_hist(i, _):
            hist_v[pl.ds(i * L, L)] = zeros
            return 0
        lax.fori_loop(0, 256, zero_hist, 0, unroll=8)

        def h1_idx(i):
            dk = _desc_key(row_v[pl.ds(i * L, L)])
            d = lax.shift_right_logical(dk, 24)
            return d * L + lane

        def h1(i, oidx):
            nxt = h1_idx(i + 1)
            plsc.addupdate_scatter(hist_v, [oidx], ones)
            return nxt
        oidx_l = lax.fori_loop(0, N // L - 1, h1, h1_idx(0), unroll=4)
        plsc.addupdate_scatter(hist_v, [oidx_l], ones)

        def coarse_sum(j, _):
            acc = zeros
            for jj in range(16):
                acc = acc + hist_v[pl.ds((j * 16 + jj) * L, L)]
            off_v[pl.ds(j * L, L)] = acc
            return 0
        lax.fori_loop(0, 16, coarse_sum, 0, unroll=2)

        def scan_c(j, carry):
            cum, jstar, cbase = carry
            t = jnp.sum(off_v[pl.ds(j * L, L)])
            ncum = cum + t
            crossed = jnp.logical_and(cum < K, ncum >= K)
            jstar = jnp.where(crossed, j, jstar)
            cbase = jnp.where(crossed, cum, cbase)
            return ncum, jstar, cbase
        _, jstar, cbase = lax.fori_loop(
            0, 16, scan_c, (jnp.int32(0), jnp.int32(0), jnp.int32(0)))

        def scan_f(i, carry):
            cum, dstar = carry
            d = jstar * 16 + i
            t = jnp.sum(hist_v[pl.ds(d * L, L)])
            ncum = cum + t
            crossed = jnp.logical_and(cum < K, ncum >= K)
            dstar = jnp.where(crossed, d, dstar)
            return ncum, dstar
        _, dstar = lax.fori_loop(0, 16, scan_f, (cbase, jnp.int32(0)))

        def keep_scan(d, acc):
            h = hist_v[pl.ds(d * L, L)]
            hist_v[pl.ds(d * L, L)] = zeros
            return acc + h * (d <= dstar).astype(jnp.int32)
        hkeep = lax.fori_loop(0, 256, keep_scan, zeros, unroll=4)
        base = plsc.cumsum(hkeep) - hkeep
        m_cnt = jnp.sum(hkeep)

        def key_msk(i):
            dk = _desc_key(row_v[pl.ds(i * L, L)])
            d = lax.shift_right_logical(dk, 24)
            return dk, d <= dstar

        def compact(i, carry):
            offv, dk, msk = carry
            dk_n, msk_n = key_msk(i + 1)
            plsc.store_scatter(a_v, [offv], dk, mask=msk)
            return offv + msk.astype(jnp.int32), dk_n, msk_n
        dk0, msk0 = key_msk(0)
        offv, dk_l, msk_l = lax.fori_loop(
            0, N // L - 1, compact, (base, dk0, msk0), unroll=2)
        plsc.store_scatter(a_v, [offv], dk_l, mask=msk_l)
        a_v[pl.ds(m_cnt, L)] = fill
        t_blk = lax.shift_right_logical(m_cnt + (L - 1), 4)
        lane_t = lane * t_blk

        def oidx_of(src, i, shift):
            k = plsc.load_gather(src, [lane_t + i])
            if src is row_v:
                k = lax.bitcast_convert_type(k, jnp.int32)
            d = lax.shift_right_logical(k, shift) & 255
            return k, d * L + lane

        def hist_pass(src, shift):
            def hp(i, carry):
                _, oidx = carry
                nxt = oidx_of(src, i + 1, shift)
                plsc.addupdate_scatter(hist_v, [oidx], ones)
                return nxt
            _, oidx_l = lax.fori_loop(0, t_blk - 1, hp, oidx_of(src, 0, shift))
            plsc.addupdate_scatter(hist_v, [oidx_l], ones)

        def offs_pass():
            def offs(d, carry):
                h = hist_v[pl.ds(d * L, L)]
                hist_v[pl.ds(d * L, L)] = zeros
                incl = plsc.cumsum(h)
                off_v[pl.ds(d * L, L)] = incl - h + carry
                return carry + jnp.sum(h)
            lax.fori_loop(0, 256, offs, jnp.int32(0), unroll=8)

        def perm_pass(src, dst, shift):
            def emit(k, oidx):
                o = plsc.load_gather(off_v, [oidx])
                plsc.store_scatter(off_v, [oidx], o + 1)
                if dst is row_v:
                    k = lax.bitcast_convert_type(k, jnp.float32)
                plsc.store_scatter(dst, [o], k)

            def perm(i, carry):
                k, oidx = carry
                nxt = oidx_of(src, i + 1, shift)
                emit(k, oidx)
                return nxt
            k_l, oidx_l = lax.fori_loop(0, t_blk - 1, perm,
                                        oidx_of(src, 0, shift))
            emit(k_l, oidx_l)

        hist_pass(a_v, 8)
        offs_pass()
        perm_pass(a_v, row_v, 8)
        hist_pass(row_v, 16)
        offs_pass()
        perm_pass(row_v, a_v, 16)
        hist_pass(a_v, 24)
        offs_pass()

        def pf_of(i):
            k, oidx = oidx_of(a_v, i, 24)
            return _key_to_f32(k), oidx

        def emitf(f, oidx):
            o = plsc.load_gather(off_v, [oidx])
            plsc.store_scatter(off_v, [oidx], o + 1)
            plsc.store_scatter(out_v, [o], f, mask=o < K)

        def permf(i, carry):
            f, oidx = carry
            nxt = pf_of(i + 1)
            emitf(f, oidx)
            return nxt
        f_l, oidx_l = lax.fori_loop(0, t_blk - 1, permf, pf_of(0))
        emitf(f_l, oidx_l)

        pltpu.sync_copy(out_v, o_hbm.at[row])

    copy_in(0, row0_v, sem0).start()

    def do_pair(rr, _):
        r0 = 2 * rr
        copy_in(r0, row0_v, sem0).wait()
        copy_in(r0 + 1, row1_v, sem1).start()
        compute_row(base_row + r0, row0_v)
        copy_in(jnp.minimum(r0 + 2, ROWS_PER_W - 1), row0_v, sem0).start()
        copy_in(r0 + 1, row1_v, sem1).wait()
        compute_row(base_row + r0 + 1, row1_v)
        return 0

    lax.fori_loop(0, ROWS_PER_W // 2, do_pair, 0)
    copy_in(ROWS_PER_W - 1, row0_v, sem0).wait()


@jax.jit
def kernel(input):
    b, h, n = input.shape
    x = input.reshape(ROWS, N)
    out = pl.kernel(
        _sc_body,
        out_type=jax.ShapeDtypeStruct((ROWS, K), jnp.float32),
        mesh=plsc.VectorSubcoreMesh(core_axis_name="c", subcore_axis_name="s"),
        compiler_params=pltpu.CompilerParams(needs_layout_passes=False),
        scratch_types=[
            pltpu.VMEM((N + L,), jnp.float32),
            pltpu.VMEM((N + L,), jnp.float32),
            pltpu.VMEM((N + L,), jnp.int32),
            pltpu.VMEM((4096,), jnp.int32),
            pltpu.VMEM((4096,), jnp.int32),
            pltpu.VMEM((K,), jnp.float32),
            pltpu.SemaphoreType.DMA,
            pltpu.SemaphoreType.DMA,
        ],
    )(x)
    return out.reshape(b, h, K)


# speedup vs baseline: 2.1488x; 1.1039x over previous
import jax
import jax.numpy as jnp
from jax import lax
from jax.experimental import pallas as pl
from jax.experimental.pallas import tpu as pltpu
from jax.experimental.pallas import tpu_sc as plsc

NC = 2
NS = 16
L = 16
NW = NC * NS

N = 32768
K = N // 4
ROWS = 2048
ROWS_PER_W = ROWS // NW

MININT = -(2**31)


def _desc_key(v):
    u = lax.bitcast_convert_type(v, jnp.int32)
    m = lax.shift_right_arithmetic(u, 31)
    a = u ^ (m | MININT)
    return ~a


def _key_to_f32(k):
    a = ~k
    u = jnp.where(a < 0, a ^ MININT, ~a)
    return lax.bitcast_convert_type(u, jnp.float32)


def _sc_body(x_hbm, o_hbm, row0_v, row1_v, a_v, hist_v, hist2_v, off_v,
             out_v, sem0, sem1):
    wid = lax.axis_index("s") * NC + lax.axis_index("c")
    lane = lax.iota(jnp.int32, L)
    ones = jnp.ones((L,), jnp.int32)
    zeros = jnp.zeros((L,), jnp.int32)
    fill = jnp.full((L,), -1, jnp.int32)
    base_row = wid * ROWS_PER_W

    def copy_in(r, buf, sem):
        return pltpu.make_async_copy(
            x_hbm.at[base_row + r], buf.at[pl.ds(0, N)], sem)

    def h1_idx(row_v, i):
        dk = _desc_key(row_v[pl.ds(i * L, L)])
        d = lax.shift_right_logical(dk, 24)
        return d * L + lane

    def h1_loop(row_v, hist):
        def h1(i, oidx):
            nxt = h1_idx(row_v, i + 1)
            plsc.addupdate_scatter(hist, [oidx], ones)
            return nxt
        oidx_l = lax.fori_loop(0, N // L - 1, h1, h1_idx(row_v, 0), unroll=4)
        plsc.addupdate_scatter(hist, [oidx_l], ones)

    def scans(hist):
        def coarse_sum(j, _):
            acc = zeros
            for jj in range(16):
                acc = acc + hist[pl.ds((j * 16 + jj) * L, L)]
            off_v[pl.ds(j * L, L)] = acc
            return 0
        lax.fori_loop(0, 16, coarse_sum, 0, unroll=2)

        def scan_c(j, carry):
            cum, jstar, cbase = carry
            t = jnp.sum(off_v[pl.ds(j * L, L)])
            ncum = cum + t
            crossed = jnp.logical_and(cum < K, ncum >= K)
            jstar = jnp.where(crossed, j, jstar)
            cbase = jnp.where(crossed, cum, cbase)
            return ncum, jstar, cbase
        _, jstar, cbase = lax.fori_loop(
            0, 16, scan_c, (jnp.int32(0), jnp.int32(0), jnp.int32(0)))

        def scan_f(i, carry):
            cum, dstar = carry
            d = jstar * 16 + i
            t = jnp.sum(hist[pl.ds(d * L, L)])
            ncum = cum + t
            crossed = jnp.logical_and(cum < K, ncum >= K)
            dstar = jnp.where(crossed, d, dstar)
            return ncum, dstar
        _, dstar = lax.fori_loop(0, 16, scan_f, (cbase, jnp.int32(0)))

        def keep_scan(d, acc):
            h = hist[pl.ds(d * L, L)]
            hist[pl.ds(d * L, L)] = zeros
            return acc + h * (d <= dstar).astype(jnp.int32)
        hkeep = lax.fori_loop(0, 256, keep_scan, zeros, unroll=4)
        base = plsc.cumsum(hkeep) - hkeep
        m_cnt = jnp.sum(hkeep)
        return dstar, base, m_cnt

    def compact_loop(row_v, dstar, base, m_cnt, other=None):
        def key_msk(i):
            dk = _desc_key(row_v[pl.ds(i * L, L)])
            d = lax.shift_right_logical(dk, 24)
            return dk, d <= dstar

        if other is None:
            def compact(i, carry):
                offv, dk, msk = carry
                dk_n, msk_n = key_msk(i + 1)
                plsc.store_scatter(a_v, [offv], dk, mask=msk)
                return offv + msk.astype(jnp.int32), dk_n, msk_n
            dk0, msk0 = key_msk(0)
            offv, dk_l, msk_l = lax.fori_loop(
                0, N // L - 1, compact, (base, dk0, msk0), unroll=2)
            plsc.store_scatter(a_v, [offv], dk_l, mask=msk_l)
        else:
            row2_v, hist2 = other

            def compact2(i, carry):
                offv, dk, msk, oidx = carry
                dk_n, msk_n = key_msk(i + 1)
                oidx_n = h1_idx(row2_v, i + 1)
                plsc.store_scatter(a_v, [offv], dk, mask=msk)
                plsc.addupdate_scatter(hist2, [oidx], ones)
                return offv + msk.astype(jnp.int32), dk_n, msk_n, oidx_n
            dk0, msk0 = key_msk(0)
            offv, dk_l, msk_l, oidx_l = lax.fori_loop(
                0, N // L - 1, compact2,
                (base, dk0, msk0, h1_idx(row2_v, 0)), unroll=2)
            plsc.store_scatter(a_v, [offv], dk_l, mask=msk_l)
            plsc.addupdate_scatter(hist2, [oidx_l], ones)
        a_v[pl.ds(m_cnt, L)] = fill

    def radix_out(row, row_v, m_cnt):
        t_blk = lax.shift_right_logical(m_cnt + (L - 1), 4)
        lane_t = lane * t_blk

        def oidx_of(src, i, shift):
            k = plsc.load_gather(src, [lane_t + i])
            if src is row_v:
                k = lax.bitcast_convert_type(k, jnp.int32)
            d = lax.shift_right_logical(k, shift) & 255
            return k, d * L + lane

        def hist_pass(src, shift):
            def hp(i, carry):
                _, oidx = carry
                nxt = oidx_of(src, i + 1, shift)
                plsc.addupdate_scatter(hist_v, [oidx], ones)
                return nxt
            _, oidx_l = lax.fori_loop(0, t_blk - 1, hp, oidx_of(src, 0, shift))
            plsc.addupdate_scatter(hist_v, [oidx_l], ones)

        def offs_pass():
            def offs(d, carry):
                h = hist_v[pl.ds(d * L, L)]
                hist_v[pl.ds(d * L, L)] = zeros
                incl = plsc.cumsum(h)
                off_v[pl.ds(d * L, L)] = incl - h + carry
                return carry + jnp.sum(h)
            lax.fori_loop(0, 256, offs, jnp.int32(0), unroll=8)

        def perm_pass(src, dst, shift):
            def emit(k, oidx):
                o = plsc.load_gather(off_v, [oidx])
                plsc.store_scatter(off_v, [oidx], o + 1)
                if dst is row_v:
                    k = lax.bitcast_convert_type(k, jnp.float32)
                plsc.store_scatter(dst, [o], k)

            def perm(i, carry):
                k, oidx = carry
                nxt = oidx_of(src, i + 1, shift)
                emit(k, oidx)
                return nxt
            k_l, oidx_l = lax.fori_loop(0, t_blk - 1, perm,
                                        oidx_of(src, 0, shift))
            emit(k_l, oidx_l)

        hist_pass(a_v, 8)
        offs_pass()
        perm_pass(a_v, row_v, 8)
        hist_pass(row_v, 16)
        offs_pass()
        perm_pass(row_v, a_v, 16)
        hist_pass(a_v, 24)
        offs_pass()

        def pf_of(i):
            k, oidx = oidx_of(a_v, i, 24)
            return _key_to_f32(k), oidx

        def emitf(f, oidx):
            o = plsc.load_gather(off_v, [oidx])
            plsc.store_scatter(off_v, [oidx], o + 1)
            plsc.store_scatter(out_v, [o], f, mask=o < K)

        def permf(i, carry):
            f, oidx = carry
            nxt = pf_of(i + 1)
            emitf(f, oidx)
            return nxt
        f_l, oidx_l = lax.fori_loop(0, t_blk - 1, permf, pf_of(0))
        emitf(f_l, oidx_l)

        pltpu.sync_copy(out_v, o_hbm.at[row])

    def zero_hists(i, _):
        hist_v[pl.ds(i * L, L)] = zeros
        hist2_v[pl.ds(i * L, L)] = zeros
        return 0
    lax.fori_loop(0, 256, zero_hists, 0, unroll=8)

    copy_in(0, row0_v, sem0).start()

    def do_pair(rr, _):
        r0 = 2 * rr
        copy_in(r0, row0_v, sem0).wait()
        copy_in(r0 + 1, row1_v, sem1).start()
        h1_loop(row0_v, hist_v)
        dstar0, base0, m0 = scans(hist_v)
        copy_in(r0 + 1, row1_v, sem1).wait()
        compact_loop(row0_v, dstar0, base0, m0, other=(row1_v, hist2_v))
        radix_out(base_row + r0, row0_v, m0)
        copy_in(jnp.minimum(r0 + 2, ROWS_PER_W - 1), row0_v, sem0).start()
        dstar1, base1, m1 = scans(hist2_v)
        compact_loop(row1_v, dstar1, base1, m1)
        radix_out(base_row + r0 + 1, row1_v, m1)
        return 0

    lax.fori_loop(0, ROWS_PER_W // 2, do_pair, 0)
    copy_in(ROWS_PER_W - 1, row0_v, sem0).wait()


@jax.jit
def kernel(input):
    b, h, n = input.shape
    x = input.reshape(ROWS, N)
    out = pl.kernel(
        _sc_body,
        out_type=jax.ShapeDtypeStruct((ROWS, K), jnp.float32),
        mesh=plsc.VectorSubcoreMesh(core_axis_name="c", subcore_axis_name="s"),
        compiler_params=pltpu.CompilerParams(needs_layout_passes=False),
        scratch_types=[
            pltpu.VMEM((N + L,), jnp.float32),
            pltpu.VMEM((N + L,), jnp.float32),
            pltpu.VMEM((N + L,), jnp.int32),
            pltpu.VMEM((4096,), jnp.int32),
            pltpu.VMEM((4096,), jnp.int32),
            pltpu.VMEM((4096,), jnp.int32),
            pltpu.VMEM((K,), jnp.float32),
            pltpu.SemaphoreType.DMA,
            pltpu.SemaphoreType.DMA,
        ],
    )(x)
    return out.reshape(b, h, K)


# speedup vs baseline: 2.1613x; 1.0058x over previous
import jax
import jax.numpy as jnp
from jax import lax
from jax.experimental import pallas as pl
from jax.experimental.pallas import tpu as pltpu
from jax.experimental.pallas import tpu_sc as plsc

NC = 2
NS = 16
L = 16
NW = NC * NS

N = 32768
K = N // 4
ROWS = 2048
ROWS_PER_W = ROWS // NW

MININT = -(2**31)


def _desc_key(v):
    u = lax.bitcast_convert_type(v, jnp.int32)
    m = lax.shift_right_arithmetic(u, 31)
    a = u ^ (m | MININT)
    return ~a


def _key_to_f32(k):
    a = ~k
    u = jnp.where(a < 0, a ^ MININT, ~a)
    return lax.bitcast_convert_type(u, jnp.float32)


def _sc_body(x_hbm, o_hbm, row0_v, row1_v, a_v, hist_v, hist2_v, off_v,
             out_v, sem0, sem1):
    wid = lax.axis_index("s") * NC + lax.axis_index("c")
    lane = lax.iota(jnp.int32, L)
    ones = jnp.ones((L,), jnp.int32)
    zeros = jnp.zeros((L,), jnp.int32)
    fill = jnp.full((L,), -1, jnp.int32)
    base_row = wid * ROWS_PER_W

    def copy_in(r, buf, sem):
        return pltpu.make_async_copy(
            x_hbm.at[base_row + r], buf.at[pl.ds(0, N)], sem)

    def h1_idx(row_v, i):
        dk = _desc_key(row_v[pl.ds(i * L, L)])
        d = lax.shift_right_logical(dk, 24)
        return d * L + lane

    def h1_loop(row_v, hist):
        def h1(i, oidx):
            nxt = h1_idx(row_v, i + 1)
            plsc.addupdate_scatter(hist, [oidx], ones)
            return nxt
        oidx_l = lax.fori_loop(0, N // L - 1, h1, h1_idx(row_v, 0), unroll=8)
        plsc.addupdate_scatter(hist, [oidx_l], ones)

    def scans(hist):
        def coarse_sum(j, _):
            acc = zeros
            for jj in range(16):
                acc = acc + hist[pl.ds((j * 16 + jj) * L, L)]
            off_v[pl.ds(j * L, L)] = acc
            return 0
        lax.fori_loop(0, 16, coarse_sum, 0, unroll=2)

        def scan_c(j, carry):
            cum, jstar, cbase = carry
            t = jnp.sum(off_v[pl.ds(j * L, L)])
            ncum = cum + t
            crossed = jnp.logical_and(cum < K, ncum >= K)
            jstar = jnp.where(crossed, j, jstar)
            cbase = jnp.where(crossed, cum, cbase)
            return ncum, jstar, cbase
        _, jstar, cbase = lax.fori_loop(
            0, 16, scan_c, (jnp.int32(0), jnp.int32(0), jnp.int32(0)))

        def scan_f(i, carry):
            cum, dstar = carry
            d = jstar * 16 + i
            t = jnp.sum(hist[pl.ds(d * L, L)])
            ncum = cum + t
            crossed = jnp.logical_and(cum < K, ncum >= K)
            dstar = jnp.where(crossed, d, dstar)
            return ncum, dstar
        _, dstar = lax.fori_loop(0, 16, scan_f, (cbase, jnp.int32(0)))

        def keep_scan(d, acc):
            h = hist[pl.ds(d * L, L)]
            hist[pl.ds(d * L, L)] = zeros
            return acc + h * (d <= dstar).astype(jnp.int32)
        hkeep = lax.fori_loop(0, 256, keep_scan, zeros, unroll=4)
        base = plsc.cumsum(hkeep) - hkeep
        m_cnt = jnp.sum(hkeep)
        return dstar, base, m_cnt

    def compact_loop(row_v, dstar, base, m_cnt, other=None):
        def key_msk(i):
            dk = _desc_key(row_v[pl.ds(i * L, L)])
            d = lax.shift_right_logical(dk, 24)
            return dk, d <= dstar

        if other is None:
            def compact(i, carry):
                offv, dk, msk = carry
                dk_n, msk_n = key_msk(i + 1)
                plsc.store_scatter(a_v, [offv], dk, mask=msk)
                return offv + msk.astype(jnp.int32), dk_n, msk_n
            dk0, msk0 = key_msk(0)
            offv, dk_l, msk_l = lax.fori_loop(
                0, N // L - 1, compact, (base, dk0, msk0), unroll=4)
            plsc.store_scatter(a_v, [offv], dk_l, mask=msk_l)
        else:
            row2_v, hist2 = other

            def compact2(i, carry):
                offv, dk, msk, oidx = carry
                dk_n, msk_n = key_msk(i + 1)
                oidx_n = h1_idx(row2_v, i + 1)
                plsc.store_scatter(a_v, [offv], dk, mask=msk)
                plsc.addupdate_scatter(hist2, [oidx], ones)
                return offv + msk.astype(jnp.int32), dk_n, msk_n, oidx_n
            dk0, msk0 = key_msk(0)
            offv, dk_l, msk_l, oidx_l = lax.fori_loop(
                0, N // L - 1, compact2,
                (base, dk0, msk0, h1_idx(row2_v, 0)), unroll=4)
            plsc.store_scatter(a_v, [offv], dk_l, mask=msk_l)
            plsc.addupdate_scatter(hist2, [oidx_l], ones)
        a_v[pl.ds(m_cnt, L)] = fill

    def radix_out(row, row_v, m_cnt):
        t_blk = lax.shift_right_logical(m_cnt + (L - 1), 4)
        lane_t = lane * t_blk

        def oidx_of(src, i, shift):
            k = plsc.load_gather(src, [lane_t + i])
            if src is row_v:
                k = lax.bitcast_convert_type(k, jnp.int32)
            d = lax.shift_right_logical(k, shift) & 255
            return k, d * L + lane

        def hist_pass(src, shift):
            def hp(i, carry):
                _, oidx = carry
                nxt = oidx_of(src, i + 1, shift)
                plsc.addupdate_scatter(hist_v, [oidx], ones)
                return nxt
            _, oidx_l = lax.fori_loop(0, t_blk - 1, hp, oidx_of(src, 0, shift))
            plsc.addupdate_scatter(hist_v, [oidx_l], ones)

        def offs_pass():
            def offs(d, carry):
                h = hist_v[pl.ds(d * L, L)]
                hist_v[pl.ds(d * L, L)] = zeros
                incl = plsc.cumsum(h)
                off_v[pl.ds(d * L, L)] = incl - h + carry
                return carry + jnp.sum(h)
            lax.fori_loop(0, 256, offs, jnp.int32(0), unroll=8)

        def perm_pass(src, dst, shift):
            def emit(k, oidx):
                o = plsc.load_gather(off_v, [oidx])
                plsc.store_scatter(off_v, [oidx], o + 1)
                if dst is row_v:
                    k = lax.bitcast_convert_type(k, jnp.float32)
                plsc.store_scatter(dst, [o], k)

            def perm(i, carry):
                k, oidx = carry
                nxt = oidx_of(src, i + 1, shift)
                emit(k, oidx)
                return nxt
            k_l, oidx_l = lax.fori_loop(0, t_blk - 1, perm,
                                        oidx_of(src, 0, shift))
            emit(k_l, oidx_l)

        hist_pass(a_v, 8)
        offs_pass()
        perm_pass(a_v, row_v, 8)
        hist_pass(row_v, 16)
        offs_pass()
        perm_pass(row_v, a_v, 16)
        hist_pass(a_v, 24)
        offs_pass()

        def pf_of(i):
            k, oidx = oidx_of(a_v, i, 24)
            return _key_to_f32(k), oidx

        def emitf(f, oidx):
            o = plsc.load_gather(off_v, [oidx])
            plsc.store_scatter(off_v, [oidx], o + 1)
            plsc.store_scatter(out_v, [o], f, mask=o < K)

        def permf(i, carry):
            f, oidx = carry
            nxt = pf_of(i + 1)
            emitf(f, oidx)
            return nxt
        f_l, oidx_l = lax.fori_loop(0, t_blk - 1, permf, pf_of(0))
        emitf(f_l, oidx_l)

        pltpu.sync_copy(out_v, o_hbm.at[row])

    def zero_hists(i, _):
        hist_v[pl.ds(i * L, L)] = zeros
        hist2_v[pl.ds(i * L, L)] = zeros
        return 0
    lax.fori_loop(0, 256, zero_hists, 0, unroll=8)

    copy_in(0, row0_v, sem0).start()

    def do_pair(rr, _):
        r0 = 2 * rr
        copy_in(r0, row0_v, sem0).wait()
        copy_in(r0 + 1, row1_v, sem1).start()
        h1_loop(row0_v, hist_v)
        dstar0, base0, m0 = scans(hist_v)
        copy_in(r0 + 1, row1_v, sem1).wait()
        compact_loop(row0_v, dstar0, base0, m0, other=(row1_v, hist2_v))
        radix_out(base_row + r0, row0_v, m0)
        copy_in(jnp.minimum(r0 + 2, ROWS_PER_W - 1), row0_v, sem0).start()
        dstar1, base1, m1 = scans(hist2_v)
        compact_loop(row1_v, dstar1, base1, m1)
        radix_out(base_row + r0 + 1, row1_v, m1)
        return 0

    lax.fori_loop(0, ROWS_PER_W // 2, do_pair, 0)
    copy_in(ROWS_PER_W - 1, row0_v, sem0).wait()


@jax.jit
def kernel(input):
    b, h, n = input.shape
    x = input.reshape(ROWS, N)
    out = pl.kernel(
        _sc_body,
        out_type=jax.ShapeDtypeStruct((ROWS, K), jnp.float32),
        mesh=plsc.VectorSubcoreMesh(core_axis_name="c", subcore_axis_name="s"),
        compiler_params=pltpu.CompilerParams(needs_layout_passes=False),
        scratch_types=[
            pltpu.VMEM((N + L,), jnp.float32),
            pltpu.VMEM((N + L,), jnp.float32),
            pltpu.VMEM((N + L,), jnp.int32),
            pltpu.VMEM((4096,), jnp.int32),
            pltpu.VMEM((4096,), jnp.int32),
            pltpu.VMEM((4096,), jnp.int32),
            pltpu.VMEM((K,), jnp.float32),
            pltpu.SemaphoreType.DMA,
            pltpu.SemaphoreType.DMA,
        ],
    )(x)
    return out.reshape(b, h, K)
